# baseline (speedup 1.0000x reference)
import functools

import jax
import jax.numpy as jnp
from jax import lax
from jax.experimental import pallas as pl
from jax.experimental.pallas import tpu as pltpu
from jax.experimental.pallas import tpu_sc as plsc

CHUNK = 128


def _sc_gather(dish2, store2, cat2, tags3, tastes3,
               dish_table, store_table, tag_table, taste_table, cat_table):
  nc, ns = 2, 16
  nw = nc * ns
  nb = dish2.shape[0]
  assert nb % nw == 0
  cpw = nb // nw

  dd = dish_table.shape[1]
  ds_ = store_table.shape[1]
  de = tag_table.shape[1]

  mesh = plsc.VectorSubcoreMesh(core_axis_name="c", subcore_axis_name="s")

  @functools.partial(
      pl.kernel,
      mesh=mesh,
      out_type=[
          jax.ShapeDtypeStruct((nb, CHUNK, dd), jnp.float32),
          jax.ShapeDtypeStruct((nb, CHUNK, ds_), jnp.float32),
          jax.ShapeDtypeStruct((nb, CHUNK, de), jnp.float32),
          jax.ShapeDtypeStruct((10, nb, CHUNK, de), jnp.float32),
          jax.ShapeDtypeStruct((5, nb, CHUNK, de), jnp.float32),
      ],
      scratch_types=[
          pltpu.VMEM((CHUNK, 10), jnp.int32),
          pltpu.VMEM((CHUNK, 5), jnp.int32),
          pltpu.VMEM((18, CHUNK), jnp.int32),
          pltpu.VMEM((CHUNK, dd), jnp.float32),
          pltpu.VMEM((CHUNK, ds_), jnp.float32),
          pltpu.VMEM((CHUNK, de), jnp.float32),
          pltpu.VMEM((10, CHUNK, de), jnp.float32),
          pltpu.VMEM((5, CHUNK, de), jnp.float32),
          pltpu.SemaphoreType.DMA,
          pltpu.SemaphoreType.DMA,
          pltpu.SemaphoreType.DMA,
      ],
      compiler_params=pltpu.CompilerParams(use_tc_tiling_on_sc=False,
                                           needs_layout_passes=False),
  )
  def gather_kernel(dish_i, store_i, cat_i, tags_i, tastes_i,
                    dish_t, store_t, tag_t, taste_t, cat_t,
                    dish_o, store_o, cat_o, tag_o, taste_o,
                    traw, sraw, idx_v, r_dish, r_store, r_cat, r_tag, r_taste,
                    sem_i, sem_g, sem_w):
    wid = lax.axis_index("s") * nc + lax.axis_index("c")
    iota16 = lax.iota(jnp.int32, 16)
    for c in range(cpw):
      r = wid * cpw + c
      loads = [
          pltpu.async_copy(dish_i.at[r], idx_v.at[0], sem_i),
          pltpu.async_copy(store_i.at[r], idx_v.at[1], sem_i),
          pltpu.async_copy(cat_i.at[r], idx_v.at[2], sem_i),
          pltpu.async_copy(tags_i.at[pl.ds(r * CHUNK, CHUNK), :], traw,
                           sem_i),
          pltpu.async_copy(tastes_i.at[pl.ds(r * CHUNK, CHUNK), :], sraw,
                           sem_i),
      ]
      for cp in loads:
        cp.wait()
      for v in range(CHUNK // 16):
        rows = iota16 + (v * 16)
        for j in range(10):
          cols = jnp.full((16,), j, jnp.int32)
          idx_v[3 + j, pl.ds(v * 16, 16)] = plsc.load_gather(
              traw, [rows, cols])
        for j in range(5):
          cols = jnp.full((16,), j, jnp.int32)
          idx_v[13 + j, pl.ds(v * 16, 16)] = plsc.load_gather(
              sraw, [rows, cols])
      gathers = [
          pltpu.async_copy(dish_t.at[idx_v.at[0]], r_dish, sem_g),
          pltpu.async_copy(store_t.at[idx_v.at[1]], r_store, sem_g),
          pltpu.async_copy(cat_t.at[idx_v.at[2]], r_cat, sem_g),
      ]
      for j in range(10):
        gathers.append(
            pltpu.async_copy(tag_t.at[idx_v.at[3 + j]], r_tag.at[j], sem_g))
      for j in range(5):
        gathers.append(
            pltpu.async_copy(taste_t.at[idx_v.at[13 + j]], r_taste.at[j],
                             sem_g))
      for cp in gathers:
        cp.wait()
      writes = [
          pltpu.async_copy(r_dish, dish_o.at[r], sem_w),
          pltpu.async_copy(r_store, store_o.at[r], sem_w),
          pltpu.async_copy(r_cat, cat_o.at[r], sem_w),
          pltpu.async_copy(r_tag, tag_o.at[:, r], sem_w),
          pltpu.async_copy(r_taste, taste_o.at[:, r], sem_w),
      ]
      for cp in writes:
        cp.wait()

  return gather_kernel(dish2, store2, cat2, tags3, tastes3,
                       dish_table, store_table, tag_table, taste_table,
                       cat_table)


def _tc_body(dish_ref, store_ref, cat_ref, tag_ref, taste_ref,
             tags_ref, tastes_ref, day_ref,
             price_ref, ot_ref, rt_ref, loc_ref, tm_ref,
             price_W_ref, price_b_ref, ot_W_ref, ot_b_ref,
             rt_W_ref, rt_b_ref, loc_W_ref, loc_b_ref, tm_W_ref, tm_b_ref,
             day_t_ref, W1_ref, b1_ref, W2_ref, b2_ref, W3_ref, b3_ref,
             out_ref):
  f32 = jnp.float32
  dish = dish_ref[...]
  store = store_ref[...]
  cat = cat_ref[...]

  tmask = (tags_ref[...] != 0).astype(f32)
  tcnt = jnp.sum(tmask, axis=1, keepdims=True)
  tsum = tag_ref[0]
  for j in range(1, 10):
    tsum = tsum + tag_ref[j]
  tag_mean = tsum / (tcnt + 1e-8)

  smask = (tastes_ref[...] != 0).astype(f32)
  scnt = jnp.sum(smask, axis=1, keepdims=True)
  ssum = taste_ref[0]
  for j in range(1, 5):
    ssum = ssum + taste_ref[j]
  taste_mean = ssum / (scnt + 1e-8)

  price_emb = price_ref[...] * price_W_ref[...] + price_b_ref[...]
  ot_emb = ot_ref[...] * ot_W_ref[...] + ot_b_ref[...]
  rt_emb = rt_ref[...] * rt_W_ref[...] + rt_b_ref[...]
  tm_emb = tm_ref[...] * tm_W_ref[...] + tm_b_ref[...]
  loc_emb = (jnp.dot(loc_ref[...], loc_W_ref[...],
                     preferred_element_type=f32) + loc_b_ref[...])

  blk = day_ref.shape[0]
  iota7 = lax.broadcasted_iota(jnp.int32, (blk, 7), 1)
  day_oh = (day_ref[...] == iota7).astype(f32)
  day_emb = jnp.dot(day_oh, day_t_ref[...], preferred_element_type=f32)

  W1 = W1_ref[...]
  h = jnp.dot(dish, W1[0:64, :], preferred_element_type=f32)
  h = h + jnp.dot(store, W1[64:96, :], preferred_element_type=f32)
  h = h + jnp.dot(tag_mean, W1[96:112, :], preferred_element_type=f32)
  h = h + jnp.dot(taste_mean, W1[112:128, :], preferred_element_type=f32)
  h = h + jnp.dot(cat, W1[128:144, :], preferred_element_type=f32)
  h = h + jnp.dot(price_emb, W1[144:160, :], preferred_element_type=f32)
  h = h + jnp.dot(ot_emb, W1[160:168, :], preferred_element_type=f32)
  h = h + jnp.dot(rt_emb, W1[168:176, :], preferred_element_type=f32)
  h = h + jnp.dot(loc_emb, W1[176:192, :], preferred_element_type=f32)
  h = h + jnp.dot(tm_emb, W1[192:200, :], preferred_element_type=f32)
  h = h + jnp.dot(day_emb, W1[200:208, :], preferred_element_type=f32)
  h = jnp.maximum(h + b1_ref[...], 0.0)

  h = jnp.dot(h, W2_ref[...], preferred_element_type=f32) + b2_ref[...]
  h = jnp.maximum(h, 0.0)
  out = jnp.dot(h, W3_ref[...], preferred_element_type=f32) + b3_ref[...]

  nrm = jnp.sqrt(jnp.sum(out * out, axis=-1, keepdims=True))
  out_ref[...] = out / jnp.maximum(nrm, 1e-12)


def kernel(dish_id, store_id, tags, tastes, category, price, order_times,
           rating, location, time_of_day, day_of_week, dish_table,
           store_table, tag_table, taste_table, cat_table, day_table,
           price_W, price_b, ot_W, ot_b, rt_W, rt_b, loc_W, loc_b, tm_W,
           tm_b, W1, b1, W2, b2, W3, b3):
  B = dish_id.shape[0]
  nb = B // CHUNK

  i32 = jnp.int32
  dish2 = dish_id.astype(i32).reshape(nb, CHUNK)
  store2 = store_id.astype(i32).reshape(nb, CHUNK)
  cat2 = category.astype(i32).reshape(nb, CHUNK)
  tags3 = tags.astype(i32)
  tastes3 = tastes.astype(i32)

  tag_tz = tag_table.at[0].set(0.0)
  taste_tz = taste_table.at[0].set(0.0)

  dish_o, store_o, cat_o, tag_o, taste_o = _sc_gather(
      dish2, store2, cat2, tags3, tastes3,
      dish_table, store_table, tag_tz, taste_tz, cat_table)

  dish_rows = dish_o.reshape(B, 64)
  store_rows = store_o.reshape(B, 32)
  cat_rows = cat_o.reshape(B, 16)
  tag_rows = tag_o.reshape(10, B, 16)
  taste_rows = taste_o.reshape(5, B, 16)

  BLK = 1024
  grid = (B // BLK,)

  def row_spec(w):
    return pl.BlockSpec((BLK, w), lambda i: (i, 0))

  def slot_spec(n):
    return pl.BlockSpec((n, BLK, 16), lambda i: (0, i, 0))

  def full_spec(shape):
    nd = len(shape)
    return pl.BlockSpec(shape, lambda i: (0,) * nd)

  out = pl.pallas_call(
      _tc_body,
      grid=grid,
      in_specs=[
          row_spec(64), row_spec(32), row_spec(16),
          slot_spec(10), slot_spec(5),
          row_spec(10), row_spec(5), row_spec(1),
          row_spec(1), row_spec(1), row_spec(1), row_spec(2), row_spec(1),
          full_spec((1, 16)), full_spec((1, 16)),
          full_spec((1, 8)), full_spec((1, 8)),
          full_spec((1, 8)), full_spec((1, 8)),
          full_spec((2, 16)), full_spec((1, 16)),
          full_spec((1, 8)), full_spec((1, 8)),
          full_spec((7, 8)),
          full_spec((208, 128)), full_spec((1, 128)),
          full_spec((128, 64)), full_spec((1, 64)),
          full_spec((64, 64)), full_spec((1, 64)),
      ],
      out_specs=row_spec(64),
      out_shape=jax.ShapeDtypeStruct((B, 64), jnp.float32),
      compiler_params=pltpu.CompilerParams(
          dimension_semantics=("parallel",)),
  )(
      dish_rows, store_rows, cat_rows, tag_rows, taste_rows,
      tags.astype(i32), tastes.astype(i32),
      day_of_week.astype(i32).reshape(B, 1),
      price, order_times, rating, location, time_of_day,
      price_W, price_b.reshape(1, 16), ot_W, ot_b.reshape(1, 8),
      rt_W, rt_b.reshape(1, 8), loc_W, loc_b.reshape(1, 16),
      tm_W, tm_b.reshape(1, 8), day_table,
      W1, b1.reshape(1, 128), W2, b2.reshape(1, 64),
      W3, b3.reshape(1, 64),
  )
  return out


# speedup vs baseline: 1.0988x; 1.0988x over previous
import functools

import jax
import jax.numpy as jnp
from jax import lax
from jax.experimental import pallas as pl
from jax.experimental.pallas import tpu as pltpu
from jax.experimental.pallas import tpu_sc as plsc

CHUNK = 128


def _sc_gather(dish2, store2, tags, tastes,
               dish_table, store_table, tag_table, taste_table):
  nc, ns = 2, 16
  nw = nc * ns
  nb = dish2.shape[0]
  B = nb * CHUNK
  assert nb % nw == 0
  cpw = nb // nw

  dd = dish_table.shape[1]
  ds_ = store_table.shape[1]
  de = tag_table.shape[1]

  mesh = plsc.VectorSubcoreMesh(core_axis_name="c", subcore_axis_name="s")

  @functools.partial(
      pl.kernel,
      mesh=mesh,
      out_type=jax.ShapeDtypeStruct((B, 128), jnp.float32),
      scratch_types=[
          pltpu.VMEM((CHUNK, 10), jnp.int32),
          pltpu.VMEM((CHUNK, 5), jnp.int32),
          pltpu.VMEM((17, CHUNK), jnp.int32),
          pltpu.VMEM((CHUNK, dd), jnp.float32),
          pltpu.VMEM((CHUNK, ds_), jnp.float32),
          pltpu.VMEM((10, CHUNK, de), jnp.float32),
          pltpu.VMEM((5, CHUNK, de), jnp.float32),
          pltpu.VMEM((CHUNK, de), jnp.float32),
          pltpu.VMEM((CHUNK, de), jnp.float32),
          pltpu.SemaphoreType.DMA,
          pltpu.SemaphoreType.DMA,
          pltpu.SemaphoreType.DMA,
      ],
      compiler_params=pltpu.CompilerParams(use_tc_tiling_on_sc=False,
                                           needs_layout_passes=False),
  )
  def gather_kernel(dish_i, store_i, tags_i, tastes_i,
                    dish_t, store_t, tag_t, taste_t,
                    a_o,
                    traw, sraw, idx_v, r_dish, r_store, r_tag, r_taste,
                    r_tsum, r_ssum,
                    sem_i, sem_g, sem_w):
    wid = lax.axis_index("s") * nc + lax.axis_index("c")
    iota16 = lax.iota(jnp.int32, 16)
    f32 = jnp.float32
    for c in range(cpw):
      r = wid * cpw + c
      base = r * CHUNK
      loads = [
          pltpu.async_copy(dish_i.at[r], idx_v.at[0], sem_i),
          pltpu.async_copy(store_i.at[r], idx_v.at[1], sem_i),
          pltpu.async_copy(tags_i.at[pl.ds(base, CHUNK), :], traw, sem_i),
          pltpu.async_copy(tastes_i.at[pl.ds(base, CHUNK), :], sraw, sem_i),
      ]
      for cp in loads:
        cp.wait()
      for v in range(CHUNK // 16):
        rows = iota16 + (v * 16)
        for j in range(10):
          idx_v[2 + j, pl.ds(v * 16, 16)] = plsc.load_gather(
              traw, [rows, jnp.full((16,), j, jnp.int32)])
        for j in range(5):
          idx_v[12 + j, pl.ds(v * 16, 16)] = plsc.load_gather(
              sraw, [rows, jnp.full((16,), j, jnp.int32)])
      gathers = [
          pltpu.async_copy(dish_t.at[idx_v.at[0]], r_dish, sem_g),
          pltpu.async_copy(store_t.at[idx_v.at[1]], r_store, sem_g),
      ]
      for j in range(10):
        gathers.append(
            pltpu.async_copy(tag_t.at[idx_v.at[2 + j]], r_tag.at[j], sem_g))
      for j in range(5):
        gathers.append(
            pltpu.async_copy(taste_t.at[idx_v.at[12 + j]], r_taste.at[j],
                             sem_g))
      for cp in gathers:
        cp.wait()

      def sum_body(v, carry):
        rows = iota16 + v * 16
        for d in range(de):
          cols = jnp.full((16,), d, jnp.int32)
          acc = plsc.load_gather(r_tag, [jnp.zeros((16,), jnp.int32),
                                         rows, cols])
          for j in range(1, 10):
            acc = acc + plsc.load_gather(
                r_tag, [jnp.full((16,), j, jnp.int32), rows, cols])
          plsc.store_scatter(r_tsum, [rows, cols], acc)
          acc2 = plsc.load_gather(r_taste, [jnp.zeros((16,), jnp.int32),
                                            rows, cols])
          for j in range(1, 5):
            acc2 = acc2 + plsc.load_gather(
                r_taste, [jnp.full((16,), j, jnp.int32), rows, cols])
          plsc.store_scatter(r_ssum, [rows, cols], acc2)
        return carry

      lax.fori_loop(0, CHUNK // 16, sum_body, 0)

      writes = [
          pltpu.async_copy(r_dish, a_o.at[pl.ds(base, CHUNK), pl.ds(0, dd)],
                           sem_w),
          pltpu.async_copy(r_store,
                           a_o.at[pl.ds(base, CHUNK), pl.ds(dd, ds_)], sem_w),
          pltpu.async_copy(r_tsum,
                           a_o.at[pl.ds(base, CHUNK), pl.ds(96, de)], sem_w),
          pltpu.async_copy(r_ssum,
                           a_o.at[pl.ds(base, CHUNK), pl.ds(112, de)], sem_w),
      ]
      for cp in writes:
        cp.wait()

  return gather_kernel(dish2, store2, tags, tastes,
                       dish_table, store_table, tag_table, taste_table)


def _tc_body(a_ref, tags_ref, tastes_ref, cat_ref, day_ref,
             price_ref, ot_ref, rt_ref, loc_ref, tm_ref,
             cat_t_ref,
             price_W_ref, price_b_ref, ot_W_ref, ot_b_ref,
             rt_W_ref, rt_b_ref, loc_W_ref, loc_b_ref, tm_W_ref, tm_b_ref,
             day_t_ref, W1_ref, b1_ref, W2_ref, b2_ref, W3_ref, b3_ref,
             out_ref):
  f32 = jnp.float32
  blk = a_ref.shape[0]

  tmask = (tags_ref[...] != 0).astype(f32)
  tcnt = jnp.sum(tmask, axis=1, keepdims=True)
  smask = (tastes_ref[...] != 0).astype(f32)
  scnt = jnp.sum(smask, axis=1, keepdims=True)
  rt_ = 1.0 / (tcnt + 1e-8)
  rs_ = 1.0 / (scnt + 1e-8)
  lane = lax.broadcasted_iota(jnp.int32, (blk, 128), 1)
  scale = jnp.where(lane < 96, 1.0, jnp.where(lane < 112, rt_, rs_))
  a = a_ref[...] * scale

  nc_ = cat_t_ref.shape[0]
  iota_c = lax.broadcasted_iota(jnp.int32, (blk, nc_), 1)
  cat_oh = (cat_ref[...] == iota_c).astype(f32)
  cat_emb = jnp.dot(cat_oh, cat_t_ref[...], preferred_element_type=f32)

  price_emb = price_ref[...] * price_W_ref[...] + price_b_ref[...]
  ot_emb = ot_ref[...] * ot_W_ref[...] + ot_b_ref[...]
  rt_emb = rt_ref[...] * rt_W_ref[...] + rt_b_ref[...]
  tm_emb = tm_ref[...] * tm_W_ref[...] + tm_b_ref[...]
  loc_emb = (jnp.dot(loc_ref[...], loc_W_ref[...],
                     preferred_element_type=f32) + loc_b_ref[...])

  iota7 = lax.broadcasted_iota(jnp.int32, (blk, 7), 1)
  day_oh = (day_ref[...] == iota7).astype(f32)
  day_emb = jnp.dot(day_oh, day_t_ref[...], preferred_element_type=f32)

  W1 = W1_ref[...]
  h = jnp.dot(a, W1[0:128, :], preferred_element_type=f32)
  h = h + jnp.dot(cat_emb, W1[128:144, :], preferred_element_type=f32)
  h = h + jnp.dot(price_emb, W1[144:160, :], preferred_element_type=f32)
  h = h + jnp.dot(ot_emb, W1[160:168, :], preferred_element_type=f32)
  h = h + jnp.dot(rt_emb, W1[168:176, :], preferred_element_type=f32)
  h = h + jnp.dot(loc_emb, W1[176:192, :], preferred_element_type=f32)
  h = h + jnp.dot(tm_emb, W1[192:200, :], preferred_element_type=f32)
  h = h + jnp.dot(day_emb, W1[200:208, :], preferred_element_type=f32)
  h = jnp.maximum(h + b1_ref[...], 0.0)

  h = jnp.dot(h, W2_ref[...], preferred_element_type=f32) + b2_ref[...]
  h = jnp.maximum(h, 0.0)
  out = jnp.dot(h, W3_ref[...], preferred_element_type=f32) + b3_ref[...]

  nrm = jnp.sqrt(jnp.sum(out * out, axis=-1, keepdims=True))
  out_ref[...] = out / jnp.maximum(nrm, 1e-12)


def kernel(dish_id, store_id, tags, tastes, category, price, order_times,
           rating, location, time_of_day, day_of_week, dish_table,
           store_table, tag_table, taste_table, cat_table, day_table,
           price_W, price_b, ot_W, ot_b, rt_W, rt_b, loc_W, loc_b, tm_W,
           tm_b, W1, b1, W2, b2, W3, b3):
  B = dish_id.shape[0]
  nb = B // CHUNK

  i32 = jnp.int32
  dish2 = dish_id.astype(i32).reshape(nb, CHUNK)
  store2 = store_id.astype(i32).reshape(nb, CHUNK)

  tag_tz = tag_table.at[0].set(0.0)
  taste_tz = taste_table.at[0].set(0.0)

  a_packed = _sc_gather(dish2, store2, tags.astype(i32), tastes.astype(i32),
                        dish_table, store_table, tag_tz, taste_tz)

  BLK = 1024
  grid = (B // BLK,)

  def row_spec(w):
    return pl.BlockSpec((BLK, w), lambda i: (i, 0))

  def full_spec(shape):
    nd = len(shape)
    return pl.BlockSpec(shape, lambda i: (0,) * nd)

  out = pl.pallas_call(
      _tc_body,
      grid=grid,
      in_specs=[
          row_spec(128),
          row_spec(10), row_spec(5), row_spec(1), row_spec(1),
          row_spec(1), row_spec(1), row_spec(1), row_spec(2), row_spec(1),
          full_spec((1000, 16)),
          full_spec((1, 16)), full_spec((1, 16)),
          full_spec((1, 8)), full_spec((1, 8)),
          full_spec((1, 8)), full_spec((1, 8)),
          full_spec((2, 16)), full_spec((1, 16)),
          full_spec((1, 8)), full_spec((1, 8)),
          full_spec((7, 8)),
          full_spec((208, 128)), full_spec((1, 128)),
          full_spec((128, 64)), full_spec((1, 64)),
          full_spec((64, 64)), full_spec((1, 64)),
      ],
      out_specs=row_spec(64),
      out_shape=jax.ShapeDtypeStruct((B, 64), jnp.float32),
      compiler_params=pltpu.CompilerParams(
          dimension_semantics=("parallel",)),
  )(
      a_packed,
      tags.astype(i32), tastes.astype(i32),
      category.astype(i32).reshape(B, 1),
      day_of_week.astype(i32).reshape(B, 1),
      price, order_times, rating, location, time_of_day,
      cat_table,
      price_W, price_b.reshape(1, 16), ot_W, ot_b.reshape(1, 8),
      rt_W, rt_b.reshape(1, 8), loc_W, loc_b.reshape(1, 16),
      tm_W, tm_b.reshape(1, 8), day_table,
      W1, b1.reshape(1, 128), W2, b2.reshape(1, 64),
      W3, b3.reshape(1, 64),
  )
  return out
